# baseline (speedup 1.0000x reference)
import jax
import jax.numpy as jnp
from jax import lax
from jax.experimental import pallas as pl
from jax.experimental.pallas import tpu as pltpu
from jax.experimental.pallas import tpu_sc as plsc

_N = 10000
_E = 320000
_D = 128
_G = 16
_EPS = 1e-5

_NC, _NS, _L = 2, 16, 16
_NW = _NC * _NS
_C = 128
_NCH = 80
_EPW = _C * _NCH
_EPAD = _NW * _EPW
_NP = 10240
_RPS = _NP // _NS
_RB = 1000
_NRB = _N // _RB


def _fill_buf(buf, val):
    def row(i, _):
        for k in range(_D // _L):
            buf[i, pl.ds(k * _L, _L)] = jnp.full((_L,), val, jnp.float32)
        return 0
    lax.fori_loop(0, _C, row, 0)


def _zero_table(buf, table_sh, s):
    _fill_buf(buf, 0.0)
    for t in range(_RPS // _C):
        pltpu.sync_copy(buf, table_sh.at[pl.ds(s * _RPS + t * _C, _C)])


def _readout(table_sh, out_hbm, c, s, buf):
    for t in range(_RPS // _C):
        r0 = s * _RPS + t * _C
        pltpu.sync_copy(table_sh.at[pl.ds(r0, _C)], buf)
        pltpu.sync_copy(buf, out_hbm.at[c, pl.ds(r0, _C)])


def _sc_deg_body(dst_hbm, out_hbm, didx_v, buf_v, table_sh, sem):
    c = lax.axis_index("c")
    s = lax.axis_index("s")
    wid = s * _NC + c

    _zero_table(buf_v, table_sh, s)
    _fill_buf(buf_v, 1.0)
    pltpu.sync_copy(dst_hbm.at[wid], didx_v)
    plsc.subcore_barrier()

    def chunk(j, _):
        pltpu.sync_copy(buf_v, table_sh.at[didx_v.at[j]], add=True)
        return 0
    lax.fori_loop(0, _NCH, chunk, 0)
    plsc.subcore_barrier()
    _readout(table_sh, out_hbm, c, s, buf_v)


def _sc_agg_body(h_hbm, src_hbm, dst_hbm, out_hbm, sidx_v, didx_v,
                 table_sh, sem):
    c = lax.axis_index("c")
    s = lax.axis_index("s")
    wid = s * _NC + c

    def prol(zb):
        _zero_table(zb, table_sh, s)
    pl.run_scoped(prol, pltpu.VMEM((_C, _D), jnp.float32))
    pltpu.sync_copy(src_hbm.at[wid], sidx_v)
    pltpu.sync_copy(dst_hbm.at[wid], didx_v)
    plsc.subcore_barrier()

    @plsc.parallel_loop(0, _NCH, unroll=2)
    def _chunk(j):
        def inner(rv, isem):
            pltpu.async_copy(h_hbm.at[sidx_v.at[j]], rv, isem).wait()
            pltpu.sync_copy(rv, table_sh.at[didx_v.at[j]], add=True)
        pl.run_scoped(inner, pltpu.VMEM((_C, _D), jnp.float32),
                      pltpu.SemaphoreType.DMA)
    plsc.subcore_barrier()

    def epil(rb):
        _readout(table_sh, out_hbm, c, s, rb)
    pl.run_scoped(epil, pltpu.VMEM((_C, _D), jnp.float32))


_sc_mesh = plsc.VectorSubcoreMesh(
    core_axis_name="c", subcore_axis_name="s",
    num_cores=_NC, num_subcores=_NS)

_deg_call = pl.kernel(
    _sc_deg_body,
    out_type=jax.ShapeDtypeStruct((_NC, _NP, _D), jnp.float32),
    mesh=_sc_mesh,
    scratch_types=[
        pltpu.VMEM((_NCH, _C), jnp.int32),
        pltpu.VMEM((_C, _D), jnp.float32),
        pltpu.VMEM_SHARED((_NP, _D), jnp.float32),
        pltpu.SemaphoreType.DMA,
    ],
)

_agg_call = pl.kernel(
    _sc_agg_body,
    out_type=jax.ShapeDtypeStruct((_NC, _NP, _D), jnp.float32),
    mesh=_sc_mesh,
    scratch_types=[
        pltpu.VMEM((_NCH, _C), jnp.int32),
        pltpu.VMEM((_NCH, _C), jnp.int32),
        pltpu.VMEM_SHARED((_NP, _D), jnp.float32),
        pltpu.SemaphoreType.DMA,
    ],
)


def _tc_dinv_body(d0_ref, d1_ref, o_ref):
    deg = 1.0 + jnp.max(d0_ref[...] + d1_ref[...], axis=1, keepdims=True)
    o_ref[...] = jnp.broadcast_to(lax.rsqrt(deg), (_RB, _D))


_dinv_call = pl.pallas_call(
    _tc_dinv_body,
    grid=(_NRB,),
    in_specs=[
        pl.BlockSpec((_RB, _D), lambda i: (i, 0)),
        pl.BlockSpec((_RB, _D), lambda i: (i, 0)),
    ],
    out_specs=pl.BlockSpec((_RB, _D), lambda i: (i, 0)),
    out_shape=jax.ShapeDtypeStruct((_N, _D), jnp.float32),
)


def _tc_lin_body(x_ref, w_ref, dinv_ref, o_ref):
    h = jnp.dot(x_ref[...], w_ref[...], preferred_element_type=jnp.float32)
    o_ref[...] = h * dinv_ref[...]


_lin_call = pl.pallas_call(
    _tc_lin_body,
    grid=(_NRB,),
    in_specs=[
        pl.BlockSpec((_RB, _D), lambda i: (i, 0)),
        pl.BlockSpec((_D, _D), lambda i: (0, 0)),
        pl.BlockSpec((_RB, _D), lambda i: (i, 0)),
    ],
    out_specs=pl.BlockSpec((_RB, _D), lambda i: (i, 0)),
    out_shape=jax.ShapeDtypeStruct((_N, _D), jnp.float32),
)


def _layer_post(a0, a1, hp, dinv, b, g, bt):
    t = dinv * (a0 + a1 + hp) + b
    m = jnp.mean(t, axis=1, keepdims=True)
    v = jnp.mean((t - m) ** 2, axis=1, keepdims=True)
    return jnp.maximum((t - m) * lax.rsqrt(v + _EPS) * g + bt, 0.0)


def _tc_mid_body(a0_ref, a1_ref, hp_ref, dinv_ref, b_ref, g_ref, bt_ref,
                 w_ref, o_ref):
    y = _layer_post(a0_ref[...], a1_ref[...], hp_ref[...], dinv_ref[...],
                    b_ref[...], g_ref[...], bt_ref[...])
    o_ref[...] = jnp.dot(y, w_ref[...],
                         preferred_element_type=jnp.float32) * dinv_ref[...]


_row_spec = pl.BlockSpec((_RB, _D), lambda i: (i, 0))
_vec_spec = pl.BlockSpec((1, _D), lambda i: (0, 0))

_mid_call = pl.pallas_call(
    _tc_mid_body,
    grid=(_NRB,),
    in_specs=[_row_spec, _row_spec, _row_spec, _row_spec,
              _vec_spec, _vec_spec, _vec_spec,
              pl.BlockSpec((_D, _D), lambda i: (0, 0))],
    out_specs=_row_spec,
    out_shape=jax.ShapeDtypeStruct((_N, _D), jnp.float32),
)


def _tc_final_body(a0_ref, a1_ref, hp_ref, dinv_ref, b_ref, g_ref, bt_ref,
                   batch_ref, o_ref, sums, cnts):
    i = pl.program_id(0)
    y = _layer_post(a0_ref[...], a1_ref[...], hp_ref[...], dinv_ref[...],
                    b_ref[...], g_ref[...], bt_ref[...])
    bv = batch_ref[0, 0, :]
    gi = lax.broadcasted_iota(jnp.int32, (_G, _RB), 0)
    oh = (gi == bv[None, :]).astype(jnp.float32)
    ps = jnp.dot(oh, y, preferred_element_type=jnp.float32)
    pc = jnp.broadcast_to(jnp.sum(oh, axis=1, keepdims=True), (_G, _D))

    @pl.when(i == 0)
    def _():
        sums[...] = jnp.zeros((_G, _D), jnp.float32)
        cnts[...] = jnp.zeros((_G, _D), jnp.float32)

    sums[...] += ps
    cnts[...] += pc

    @pl.when(i == pl.num_programs(0) - 1)
    def _():
        o_ref[...] = sums[...] / jnp.maximum(cnts[...], 1.0)


_final_call = pl.pallas_call(
    _tc_final_body,
    grid=(_NRB,),
    in_specs=[_row_spec, _row_spec, _row_spec, _row_spec,
              _vec_spec, _vec_spec, _vec_spec,
              pl.BlockSpec((1, 1, _RB), lambda i: (i, 0, 0))],
    out_specs=pl.BlockSpec((_G, _D), lambda i: (0, 0)),
    out_shape=jax.ShapeDtypeStruct((_G, _D), jnp.float32),
    scratch_shapes=[pltpu.VMEM((_G, _D), jnp.float32),
                    pltpu.VMEM((_G, _D), jnp.float32)],
)


def kernel(x, edge_index, batch, W1, b1, g1, bt1, W2, b2, g2, bt2,
           W3, b3, g3, bt3):
    src = edge_index[0]
    dst = edge_index[1]
    pad = _EPAD - _E
    src_p = jnp.concatenate(
        [src, jnp.zeros((pad,), jnp.int32)]).reshape(_NW, _NCH, _C)
    dst_p = jnp.concatenate(
        [dst, jnp.full((pad,), _N, jnp.int32)]).reshape(_NW, _NCH, _C)

    dparts = _deg_call(dst_p)
    dinv = _dinv_call(dparts[0, :_N], dparts[1, :_N])

    b1r, g1r, bt1r = b1.reshape(1, _D), g1.reshape(1, _D), bt1.reshape(1, _D)
    b2r, g2r, bt2r = b2.reshape(1, _D), g2.reshape(1, _D), bt2.reshape(1, _D)
    b3r, g3r, bt3r = b3.reshape(1, _D), g3.reshape(1, _D), bt3.reshape(1, _D)
    batch3 = batch.reshape(_NRB, 1, _RB)

    h1 = _lin_call(x, W1, dinv)
    a = _agg_call(h1, src_p, dst_p)
    h2 = _mid_call(a[0, :_N], a[1, :_N], h1, dinv, b1r, g1r, bt1r, W2)
    a = _agg_call(h2, src_p, dst_p)
    h3 = _mid_call(a[0, :_N], a[1, :_N], h2, dinv, b2r, g2r, bt2r, W3)
    a = _agg_call(h3, src_p, dst_p)
    return _final_call(a[0, :_N], a[1, :_N], h3, dinv, b3r, g3r, bt3r, batch3)


# speedup vs baseline: 2.3848x; 2.3848x over previous
import jax
import jax.numpy as jnp
from jax import lax
from jax.experimental import pallas as pl
from jax.experimental.pallas import tpu as pltpu
from jax.experimental.pallas import tpu_sc as plsc

_N = 10000
_E = 320000
_D = 128
_G = 16
_EPS = 1e-5

_NC, _NS, _L = 2, 16, 16
_NW = _NC * _NS
_C = 128
_NCH = 80
_EPW = _C * _NCH
_EPAD = _NW * _EPW
_NP = 10240
_RPS = _NP // _NS
_RB = 1000
_NRB = _N // _RB
_K = 3


def _fill_buf(buf, val):
    def row(i, _):
        for k in range(_D // _L):
            buf[i, pl.ds(k * _L, _L)] = jnp.full((_L,), val, jnp.float32)
        return 0
    lax.fori_loop(0, _C, row, 0)


def _zero_table(buf, table_sh, s):
    _fill_buf(buf, 0.0)
    for t in range(_RPS // _C):
        pltpu.sync_copy(buf, table_sh.at[pl.ds(s * _RPS + t * _C, _C)])


def _readout(table_sh, out_hbm, c, s, buf):
    for t in range(_RPS // _C):
        r0 = s * _RPS + t * _C
        pltpu.sync_copy(table_sh.at[pl.ds(r0, _C)], buf)
        pltpu.sync_copy(buf, out_hbm.at[c, pl.ds(r0, _C)])


def _sc_deg_body(dst_hbm, out_hbm, didx_v, buf_v, table_sh, sem):
    c = lax.axis_index("c")
    s = lax.axis_index("s")
    wid = s * _NC + c

    _zero_table(buf_v, table_sh, s)
    _fill_buf(buf_v, 1.0)
    pltpu.sync_copy(dst_hbm.at[wid], didx_v)
    plsc.subcore_barrier()

    def chunk(j, _):
        pltpu.sync_copy(buf_v, table_sh.at[didx_v.at[j]], add=True)
        return 0
    lax.fori_loop(0, _NCH, chunk, 0)
    plsc.subcore_barrier()
    _readout(table_sh, out_hbm, c, s, buf_v)


def _sc_agg_body(h_hbm, src_hbm, dst_hbm, out_hbm, sidx_v, didx_v, rows_big,
                 table_sh, sem):
    c = lax.axis_index("c")
    s = lax.axis_index("s")
    wid = s * _NC + c
    buf0 = rows_big.at[pl.ds(0, _C)]

    _zero_table(buf0, table_sh, s)
    pltpu.sync_copy(src_hbm.at[wid], sidx_v)
    pltpu.sync_copy(dst_hbm.at[wid], didx_v)
    plsc.subcore_barrier()

    def chunk(j, _):
        pltpu.async_copy(h_hbm.at[sidx_v.at[j]], buf0, sem).wait()
        pltpu.sync_copy(buf0, table_sh.at[didx_v.at[j]], add=True)
        return 0
    lax.fori_loop(0, _NCH, chunk, 0)
    plsc.subcore_barrier()
    _readout(table_sh, out_hbm, c, s, buf0)


_sc_mesh = plsc.VectorSubcoreMesh(
    core_axis_name="c", subcore_axis_name="s",
    num_cores=_NC, num_subcores=_NS)

_deg_call = pl.kernel(
    _sc_deg_body,
    out_type=jax.ShapeDtypeStruct((_NC, _NP, _D), jnp.float32),
    mesh=_sc_mesh,
    scratch_types=[
        pltpu.VMEM((_NCH, _C), jnp.int32),
        pltpu.VMEM((_C, _D), jnp.float32),
        pltpu.VMEM_SHARED((_NP, _D), jnp.float32),
        pltpu.SemaphoreType.DMA,
    ],
)

_agg_call = pl.kernel(
    _sc_agg_body,
    out_type=jax.ShapeDtypeStruct((_NC, _NP, _D), jnp.float32),
    mesh=_sc_mesh,
    scratch_types=[
        pltpu.VMEM((_NCH, _C), jnp.int32),
        pltpu.VMEM((_NCH, _C), jnp.int32),
        pltpu.VMEM((_C, _D), jnp.float32),
        pltpu.VMEM_SHARED((_NP, _D), jnp.float32),
        pltpu.SemaphoreType.DMA,
    ],
)


def _tc_dinv_body(d0_ref, d1_ref, o_ref):
    deg = 1.0 + jnp.max(d0_ref[...] + d1_ref[...], axis=1, keepdims=True)
    o_ref[...] = jnp.broadcast_to(lax.rsqrt(deg), (_RB, _D))


_dinv_call = pl.pallas_call(
    _tc_dinv_body,
    grid=(_NRB,),
    in_specs=[
        pl.BlockSpec((_RB, _D), lambda i: (i, 0)),
        pl.BlockSpec((_RB, _D), lambda i: (i, 0)),
    ],
    out_specs=pl.BlockSpec((_RB, _D), lambda i: (i, 0)),
    out_shape=jax.ShapeDtypeStruct((_N, _D), jnp.float32),
)


def _tc_lin_body(x_ref, w_ref, dinv_ref, o_ref):
    h = jnp.dot(x_ref[...], w_ref[...], preferred_element_type=jnp.float32)
    o_ref[...] = h * dinv_ref[...]


_lin_call = pl.pallas_call(
    _tc_lin_body,
    grid=(_NRB,),
    in_specs=[
        pl.BlockSpec((_RB, _D), lambda i: (i, 0)),
        pl.BlockSpec((_D, _D), lambda i: (0, 0)),
        pl.BlockSpec((_RB, _D), lambda i: (i, 0)),
    ],
    out_specs=pl.BlockSpec((_RB, _D), lambda i: (i, 0)),
    out_shape=jax.ShapeDtypeStruct((_N, _D), jnp.float32),
)


def _layer_post(a0, a1, hp, dinv, b, g, bt):
    t = dinv * (a0 + a1 + hp) + b
    m = jnp.mean(t, axis=1, keepdims=True)
    v = jnp.mean((t - m) ** 2, axis=1, keepdims=True)
    return jnp.maximum((t - m) * lax.rsqrt(v + _EPS) * g + bt, 0.0)


def _tc_mid_body(a0_ref, a1_ref, hp_ref, dinv_ref, b_ref, g_ref, bt_ref,
                 w_ref, o_ref):
    y = _layer_post(a0_ref[...], a1_ref[...], hp_ref[...], dinv_ref[...],
                    b_ref[...], g_ref[...], bt_ref[...])
    o_ref[...] = jnp.dot(y, w_ref[...],
                         preferred_element_type=jnp.float32) * dinv_ref[...]


_row_spec = pl.BlockSpec((_RB, _D), lambda i: (i, 0))
_vec_spec = pl.BlockSpec((1, _D), lambda i: (0, 0))

_mid_call = pl.pallas_call(
    _tc_mid_body,
    grid=(_NRB,),
    in_specs=[_row_spec, _row_spec, _row_spec, _row_spec,
              _vec_spec, _vec_spec, _vec_spec,
              pl.BlockSpec((_D, _D), lambda i: (0, 0))],
    out_specs=_row_spec,
    out_shape=jax.ShapeDtypeStruct((_N, _D), jnp.float32),
)


def _tc_final_body(a0_ref, a1_ref, hp_ref, dinv_ref, b_ref, g_ref, bt_ref,
                   batch_ref, o_ref, sums, cnts):
    i = pl.program_id(0)
    y = _layer_post(a0_ref[...], a1_ref[...], hp_ref[...], dinv_ref[...],
                    b_ref[...], g_ref[...], bt_ref[...])
    bv = batch_ref[0, 0, :]
    gi = lax.broadcasted_iota(jnp.int32, (_G, _RB), 0)
    oh = (gi == bv[None, :]).astype(jnp.float32)
    ps = jnp.dot(oh, y, preferred_element_type=jnp.float32)
    pc = jnp.broadcast_to(jnp.sum(oh, axis=1, keepdims=True), (_G, _D))

    @pl.when(i == 0)
    def _():
        sums[...] = jnp.zeros((_G, _D), jnp.float32)
        cnts[...] = jnp.zeros((_G, _D), jnp.float32)

    sums[...] += ps
    cnts[...] += pc

    @pl.when(i == pl.num_programs(0) - 1)
    def _():
        o_ref[...] = sums[...] / jnp.maximum(cnts[...], 1.0)


_final_call = pl.pallas_call(
    _tc_final_body,
    grid=(_NRB,),
    in_specs=[_row_spec, _row_spec, _row_spec, _row_spec,
              _vec_spec, _vec_spec, _vec_spec,
              pl.BlockSpec((1, 1, _RB), lambda i: (i, 0, 0))],
    out_specs=pl.BlockSpec((_G, _D), lambda i: (0, 0)),
    out_shape=jax.ShapeDtypeStruct((_G, _D), jnp.float32),
    scratch_shapes=[pltpu.VMEM((_G, _D), jnp.float32),
                    pltpu.VMEM((_G, _D), jnp.float32)],
)


def kernel(x, edge_index, batch, W1, b1, g1, bt1, W2, b2, g2, bt2,
           W3, b3, g3, bt3):
    src = edge_index[0]
    dst = edge_index[1]
    pad = _EPAD - _E
    lanes = jnp.arange(pad, dtype=jnp.int32)
    src_p = jnp.concatenate(
        [src, lanes % _N]).reshape(_NW, _NCH, _C)
    dst_p = jnp.concatenate(
        [dst, _N + lanes % (_NP - _N)]).reshape(_NW, _NCH, _C)

    dparts = _deg_call(dst_p)
    dinv = _dinv_call(dparts[0, :_N], dparts[1, :_N])

    b1r, g1r, bt1r = b1.reshape(1, _D), g1.reshape(1, _D), bt1.reshape(1, _D)
    b2r, g2r, bt2r = b2.reshape(1, _D), g2.reshape(1, _D), bt2.reshape(1, _D)
    b3r, g3r, bt3r = b3.reshape(1, _D), g3.reshape(1, _D), bt3.reshape(1, _D)
    batch3 = batch.reshape(_NRB, 1, _RB)

    h1 = _lin_call(x, W1, dinv)
    a = _agg_call(h1, src_p, dst_p)
    h2 = _mid_call(a[0, :_N], a[1, :_N], h1, dinv, b1r, g1r, bt1r, W2)
    a = _agg_call(h2, src_p, dst_p)
    h3 = _mid_call(a[0, :_N], a[1, :_N], h2, dinv, b2r, g2r, bt2r, W3)
    a = _agg_call(h3, src_p, dst_p)
    return _final_call(a[0, :_N], a[1, :_N], h3, dinv, b3r, g3r, bt3r, batch3)


# speedup vs baseline: 2.4146x; 1.0125x over previous
import jax
import jax.numpy as jnp
from jax import lax
from jax.experimental import pallas as pl
from jax.experimental.pallas import tpu as pltpu
from jax.experimental.pallas import tpu_sc as plsc

_N = 10000
_E = 320000
_D = 128
_G = 16
_EPS = 1e-5

_NC, _NS, _L = 2, 16, 16
_NW = _NC * _NS
_C = 128
_NCH = 80
_EPW = _C * _NCH
_EPAD = _NW * _EPW
_NP = 10240
_RPS = _NP // _NS
_RB = 1000
_NRB = _N // _RB
_K = 3


def _fill_buf(buf, val):
    def row(i, _):
        for k in range(_D // _L):
            buf[i, pl.ds(k * _L, _L)] = jnp.full((_L,), val, jnp.float32)
        return 0
    lax.fori_loop(0, _C, row, 0)


def _zero_table(buf, table_sh, s):
    _fill_buf(buf, 0.0)
    for t in range(_RPS // _C):
        pltpu.sync_copy(buf, table_sh.at[pl.ds(s * _RPS + t * _C, _C)])


def _readout(table_sh, out_hbm, c, s, sem):
    for t in range(_RPS // _C):
        r0 = s * _RPS + t * _C
        pltpu.async_copy(table_sh.at[pl.ds(r0, _C)],
                         out_hbm.at[c, pl.ds(r0, _C)], sem)
    for t in range(_RPS // _C):
        r0 = s * _RPS + t * _C
        pltpu.make_async_copy(table_sh.at[pl.ds(r0, _C)],
                              out_hbm.at[c, pl.ds(r0, _C)], sem).wait()


def _sc_deg_body(dst_hbm, out_hbm, didx_v, buf_v, table_sh, sem):
    c = lax.axis_index("c")
    s = lax.axis_index("s")
    wid = s * _NC + c

    _zero_table(buf_v, table_sh, s)
    _fill_buf(buf_v, 1.0)
    pltpu.sync_copy(dst_hbm.at[wid], didx_v)
    plsc.subcore_barrier()

    def chunk(j, _):
        pltpu.sync_copy(buf_v, table_sh.at[didx_v.at[j]], add=True)
        return 0
    lax.fori_loop(0, _NCH, chunk, 0)
    plsc.subcore_barrier()
    _readout(table_sh, out_hbm, c, s, sem)


def _sc_agg_body(h_hbm, src_hbm, dst_hbm, out_hbm, sidx_v, didx_v, rows_big,
                 table_sh, sem):
    c = lax.axis_index("c")
    s = lax.axis_index("s")
    wid = s * _NC + c
    buf0 = rows_big.at[pl.ds(0, _C)]

    _zero_table(buf0, table_sh, s)
    pltpu.sync_copy(src_hbm.at[wid], sidx_v)
    pltpu.sync_copy(dst_hbm.at[wid], didx_v)
    plsc.subcore_barrier()

    def chunk(j, _):
        pltpu.async_copy(h_hbm.at[sidx_v.at[j]], buf0, sem).wait()
        pltpu.sync_copy(buf0, table_sh.at[didx_v.at[j]], add=True)
        return 0
    lax.fori_loop(0, _NCH, chunk, 0)
    plsc.subcore_barrier()
    _readout(table_sh, out_hbm, c, s, sem)


_sc_mesh = plsc.VectorSubcoreMesh(
    core_axis_name="c", subcore_axis_name="s",
    num_cores=_NC, num_subcores=_NS)

_deg_call = pl.kernel(
    _sc_deg_body,
    out_type=jax.ShapeDtypeStruct((_NC, _NP, _D), jnp.float32),
    mesh=_sc_mesh,
    scratch_types=[
        pltpu.VMEM((_NCH, _C), jnp.int32),
        pltpu.VMEM((_C, _D), jnp.float32),
        pltpu.VMEM_SHARED((_NP, _D), jnp.float32),
        pltpu.SemaphoreType.DMA,
    ],
)

_agg_call = pl.kernel(
    _sc_agg_body,
    out_type=jax.ShapeDtypeStruct((_NC, _NP, _D), jnp.float32),
    mesh=_sc_mesh,
    scratch_types=[
        pltpu.VMEM((_NCH, _C), jnp.int32),
        pltpu.VMEM((_NCH, _C), jnp.int32),
        pltpu.VMEM((_C, _D), jnp.float32),
        pltpu.VMEM_SHARED((_NP, _D), jnp.float32),
        pltpu.SemaphoreType.DMA,
    ],
)


def _tc_lin_body(x_ref, w_ref, d0_ref, d1_ref, h_ref, dinv_ref):
    deg = 1.0 + jnp.max(d0_ref[...] + d1_ref[...], axis=1, keepdims=True)
    dv = jnp.broadcast_to(lax.rsqrt(deg), (_RB, _D))
    dinv_ref[...] = dv
    h = jnp.dot(x_ref[...], w_ref[...], preferred_element_type=jnp.float32)
    h_ref[...] = h * dv


_lin_call = pl.pallas_call(
    _tc_lin_body,
    grid=(_NRB,),
    in_specs=[
        pl.BlockSpec((_RB, _D), lambda i: (i, 0)),
        pl.BlockSpec((_D, _D), lambda i: (0, 0)),
        pl.BlockSpec((_RB, _D), lambda i: (i, 0)),
        pl.BlockSpec((_RB, _D), lambda i: (i, 0)),
    ],
    out_specs=[pl.BlockSpec((_RB, _D), lambda i: (i, 0)),
               pl.BlockSpec((_RB, _D), lambda i: (i, 0))],
    out_shape=[jax.ShapeDtypeStruct((_N, _D), jnp.float32),
               jax.ShapeDtypeStruct((_N, _D), jnp.float32)],
)


def _layer_post(a0, a1, hp, dinv, b, g, bt):
    t = dinv * (a0 + a1 + hp) + b
    m = jnp.mean(t, axis=1, keepdims=True)
    v = jnp.mean((t - m) ** 2, axis=1, keepdims=True)
    return jnp.maximum((t - m) * lax.rsqrt(v + _EPS) * g + bt, 0.0)


def _tc_mid_body(a0_ref, a1_ref, hp_ref, dinv_ref, b_ref, g_ref, bt_ref,
                 w_ref, o_ref):
    y = _layer_post(a0_ref[...], a1_ref[...], hp_ref[...], dinv_ref[...],
                    b_ref[...], g_ref[...], bt_ref[...])
    o_ref[...] = jnp.dot(y, w_ref[...],
                         preferred_element_type=jnp.float32) * dinv_ref[...]


_row_spec = pl.BlockSpec((_RB, _D), lambda i: (i, 0))
_vec_spec = pl.BlockSpec((1, _D), lambda i: (0, 0))

_mid_call = pl.pallas_call(
    _tc_mid_body,
    grid=(_NRB,),
    in_specs=[_row_spec, _row_spec, _row_spec, _row_spec,
              _vec_spec, _vec_spec, _vec_spec,
              pl.BlockSpec((_D, _D), lambda i: (0, 0))],
    out_specs=_row_spec,
    out_shape=jax.ShapeDtypeStruct((_N, _D), jnp.float32),
)


def _tc_final_body(a0_ref, a1_ref, hp_ref, dinv_ref, b_ref, g_ref, bt_ref,
                   batch_ref, o_ref, sums, cnts):
    i = pl.program_id(0)
    y = _layer_post(a0_ref[...], a1_ref[...], hp_ref[...], dinv_ref[...],
                    b_ref[...], g_ref[...], bt_ref[...])
    bv = batch_ref[0, 0, :]
    gi = lax.broadcasted_iota(jnp.int32, (_G, _RB), 0)
    oh = (gi == bv[None, :]).astype(jnp.float32)
    ps = jnp.dot(oh, y, preferred_element_type=jnp.float32)
    pc = jnp.broadcast_to(jnp.sum(oh, axis=1, keepdims=True), (_G, _D))

    @pl.when(i == 0)
    def _():
        sums[...] = jnp.zeros((_G, _D), jnp.float32)
        cnts[...] = jnp.zeros((_G, _D), jnp.float32)

    sums[...] += ps
    cnts[...] += pc

    @pl.when(i == pl.num_programs(0) - 1)
    def _():
        o_ref[...] = sums[...] / jnp.maximum(cnts[...], 1.0)


_final_call = pl.pallas_call(
    _tc_final_body,
    grid=(_NRB,),
    in_specs=[_row_spec, _row_spec, _row_spec, _row_spec,
              _vec_spec, _vec_spec, _vec_spec,
              pl.BlockSpec((1, 1, _RB), lambda i: (i, 0, 0))],
    out_specs=pl.BlockSpec((_G, _D), lambda i: (0, 0)),
    out_shape=jax.ShapeDtypeStruct((_G, _D), jnp.float32),
    scratch_shapes=[pltpu.VMEM((_G, _D), jnp.float32),
                    pltpu.VMEM((_G, _D), jnp.float32)],
)


def kernel(x, edge_index, batch, W1, b1, g1, bt1, W2, b2, g2, bt2,
           W3, b3, g3, bt3):
    src = edge_index[0]
    dst = edge_index[1]
    pad = _EPAD - _E
    lanes = jnp.arange(pad, dtype=jnp.int32)
    src_p = jnp.concatenate(
        [src, lanes % _N]).reshape(_NW, _NCH, _C)
    dst_p = jnp.concatenate(
        [dst, _N + lanes % (_NP - _N)]).reshape(_NW, _NCH, _C)

    dparts = _deg_call(dst_p)

    b1r, g1r, bt1r = b1.reshape(1, _D), g1.reshape(1, _D), bt1.reshape(1, _D)
    b2r, g2r, bt2r = b2.reshape(1, _D), g2.reshape(1, _D), bt2.reshape(1, _D)
    b3r, g3r, bt3r = b3.reshape(1, _D), g3.reshape(1, _D), bt3.reshape(1, _D)
    batch3 = batch.reshape(_NRB, 1, _RB)

    h1, dinv = _lin_call(x, W1, dparts[0, :_N], dparts[1, :_N])
    a = _agg_call(h1, src_p, dst_p)
    h2 = _mid_call(a[0, :_N], a[1, :_N], h1, dinv, b1r, g1r, bt1r, W2)
    a = _agg_call(h2, src_p, dst_p)
    h3 = _mid_call(a[0, :_N], a[1, :_N], h2, dinv, b2r, g2r, bt2r, W3)
    a = _agg_call(h3, src_p, dst_p)
    return _final_call(a[0, :_N], a[1, :_N], h3, dinv, b3r, g3r, bt3r, batch3)


# speedup vs baseline: 2.5093x; 1.0392x over previous
import jax
import jax.numpy as jnp
from jax import lax
from jax.experimental import pallas as pl
from jax.experimental.pallas import tpu as pltpu
from jax.experimental.pallas import tpu_sc as plsc

_N = 10000
_E = 320000
_D = 128
_G = 16
_EPS = 1e-5

_NC, _NS, _L = 2, 16, 16
_NW = _NC * _NS
_C = 128
_NCH = 80
_EPW = _C * _NCH
_EPAD = _NW * _EPW
_NP = 10240
_RPS = _NP // _NS
_RB = 1000
_NRB = _N // _RB
_K = 3


def _fill_buf(buf, val):
    def row(i, _):
        for k in range(_D // _L):
            buf[i, pl.ds(k * _L, _L)] = jnp.full((_L,), val, jnp.float32)
        return 0
    lax.fori_loop(0, _C, row, 0)


def _zero_table(buf, table_sh, s):
    _fill_buf(buf, 0.0)
    for t in range(_RPS // _C):
        pltpu.sync_copy(buf, table_sh.at[pl.ds(s * _RPS + t * _C, _C)])


def _readout(table_sh, out_hbm, c, s, sem):
    for t in range(_RPS // _C):
        r0 = s * _RPS + t * _C
        pltpu.async_copy(table_sh.at[pl.ds(r0, _C)],
                         out_hbm.at[c, pl.ds(r0, _C)], sem)
    for t in range(_RPS // _C):
        r0 = s * _RPS + t * _C
        pltpu.make_async_copy(table_sh.at[pl.ds(r0, _C)],
                              out_hbm.at[c, pl.ds(r0, _C)], sem).wait()


_DW = 64


def _fill_buf_w(buf, val):
    def row(i, _):
        for k in range(_DW // _L):
            buf[i, pl.ds(k * _L, _L)] = jnp.full((_L,), val, jnp.float32)
        return 0
    lax.fori_loop(0, _C, row, 0)


def _sc_deg_body(dst_hbm, out_hbm, didx_v, buf_v, table_sh, sem):
    c = lax.axis_index("c")
    s = lax.axis_index("s")
    wid = s * _NC + c

    _fill_buf_w(buf_v, 0.0)
    for t in range(_RPS // _C):
        pltpu.sync_copy(buf_v, table_sh.at[pl.ds(s * _RPS + t * _C, _C)])
    _fill_buf_w(buf_v, 1.0)
    pltpu.sync_copy(dst_hbm.at[wid], didx_v)
    plsc.subcore_barrier()

    def chunk(j, _):
        pltpu.sync_copy(buf_v, table_sh.at[didx_v.at[j]], add=True)
        return 0
    lax.fori_loop(0, _NCH, chunk, 0)
    plsc.subcore_barrier()
    _readout(table_sh, out_hbm, c, s, sem)


def _sc_agg_body(h_hbm, src_hbm, dst_hbm, out_hbm, sidx_v, didx_v, rows_big,
                 table_sh, sem):
    c = lax.axis_index("c")
    s = lax.axis_index("s")
    wid = s * _NC + c
    buf0 = rows_big.at[pl.ds(0, _C)]

    _zero_table(buf0, table_sh, s)
    pltpu.sync_copy(src_hbm.at[wid], sidx_v)
    pltpu.sync_copy(dst_hbm.at[wid], didx_v)
    plsc.subcore_barrier()

    def chunk(j, _):
        pltpu.async_copy(h_hbm.at[sidx_v.at[j]], buf0, sem).wait()
        pltpu.sync_copy(buf0, table_sh.at[didx_v.at[j]], add=True)
        return 0
    lax.fori_loop(0, _NCH, chunk, 0)
    plsc.subcore_barrier()
    _readout(table_sh, out_hbm, c, s, sem)


_sc_mesh = plsc.VectorSubcoreMesh(
    core_axis_name="c", subcore_axis_name="s",
    num_cores=_NC, num_subcores=_NS)

_deg_call = pl.kernel(
    _sc_deg_body,
    out_type=jax.ShapeDtypeStruct((_NC, _NP, _DW), jnp.float32),
    mesh=_sc_mesh,
    scratch_types=[
        pltpu.VMEM((_NCH, _C), jnp.int32),
        pltpu.VMEM((_C, _DW), jnp.float32),
        pltpu.VMEM_SHARED((_NP, _DW), jnp.float32),
        pltpu.SemaphoreType.DMA,
    ],
)

_agg_call = pl.kernel(
    _sc_agg_body,
    out_type=jax.ShapeDtypeStruct((_NC, _NP, _D), jnp.float32),
    mesh=_sc_mesh,
    scratch_types=[
        pltpu.VMEM((_NCH, _C), jnp.int32),
        pltpu.VMEM((_NCH, _C), jnp.int32),
        pltpu.VMEM((_C, _D), jnp.float32),
        pltpu.VMEM_SHARED((_NP, _D), jnp.float32),
        pltpu.SemaphoreType.DMA,
    ],
)


def _tc_lin_body(x_ref, w_ref, d0_ref, d1_ref, h_ref, dinv_ref):
    deg = 1.0 + jnp.max(d0_ref[...] + d1_ref[...], axis=1, keepdims=True)
    dv = jnp.broadcast_to(lax.rsqrt(deg), (_RB, _D))
    dinv_ref[...] = dv
    h = jnp.dot(x_ref[...], w_ref[...], preferred_element_type=jnp.float32)
    h_ref[...] = h * dv


_lin_call = pl.pallas_call(
    _tc_lin_body,
    grid=(_NRB,),
    in_specs=[
        pl.BlockSpec((_RB, _D), lambda i: (i, 0)),
        pl.BlockSpec((_D, _D), lambda i: (0, 0)),
        pl.BlockSpec((_RB, _DW), lambda i: (i, 0)),
        pl.BlockSpec((_RB, _DW), lambda i: (i, 0)),
    ],
    out_specs=[pl.BlockSpec((_RB, _D), lambda i: (i, 0)),
               pl.BlockSpec((_RB, _D), lambda i: (i, 0))],
    out_shape=[jax.ShapeDtypeStruct((_N, _D), jnp.float32),
               jax.ShapeDtypeStruct((_N, _D), jnp.float32)],
)


def _layer_post(a0, a1, hp, dinv, b, g, bt):
    t = dinv * (a0 + a1 + hp) + b
    m = jnp.mean(t, axis=1, keepdims=True)
    v = jnp.mean((t - m) ** 2, axis=1, keepdims=True)
    return jnp.maximum((t - m) * lax.rsqrt(v + _EPS) * g + bt, 0.0)


def _tc_mid_body(a0_ref, a1_ref, hp_ref, dinv_ref, b_ref, g_ref, bt_ref,
                 w_ref, o_ref):
    y = _layer_post(a0_ref[...], a1_ref[...], hp_ref[...], dinv_ref[...],
                    b_ref[...], g_ref[...], bt_ref[...])
    o_ref[...] = jnp.dot(y, w_ref[...],
                         preferred_element_type=jnp.float32) * dinv_ref[...]


_row_spec = pl.BlockSpec((_RB, _D), lambda i: (i, 0))
_vec_spec = pl.BlockSpec((1, _D), lambda i: (0, 0))

_mid_call = pl.pallas_call(
    _tc_mid_body,
    grid=(_NRB,),
    in_specs=[_row_spec, _row_spec, _row_spec, _row_spec,
              _vec_spec, _vec_spec, _vec_spec,
              pl.BlockSpec((_D, _D), lambda i: (0, 0))],
    out_specs=_row_spec,
    out_shape=jax.ShapeDtypeStruct((_N, _D), jnp.float32),
)


def _tc_final_body(a0_ref, a1_ref, hp_ref, dinv_ref, b_ref, g_ref, bt_ref,
                   batch_ref, o_ref, sums, cnts):
    i = pl.program_id(0)
    y = _layer_post(a0_ref[...], a1_ref[...], hp_ref[...], dinv_ref[...],
                    b_ref[...], g_ref[...], bt_ref[...])
    bv = batch_ref[0, 0, :]
    gi = lax.broadcasted_iota(jnp.int32, (_G, _RB), 0)
    oh = (gi == bv[None, :]).astype(jnp.float32)
    ps = jnp.dot(oh, y, preferred_element_type=jnp.float32)
    pc = jnp.broadcast_to(jnp.sum(oh, axis=1, keepdims=True), (_G, _D))

    @pl.when(i == 0)
    def _():
        sums[...] = jnp.zeros((_G, _D), jnp.float32)
        cnts[...] = jnp.zeros((_G, _D), jnp.float32)

    sums[...] += ps
    cnts[...] += pc

    @pl.when(i == pl.num_programs(0) - 1)
    def _():
        o_ref[...] = sums[...] / jnp.maximum(cnts[...], 1.0)


_final_call = pl.pallas_call(
    _tc_final_body,
    grid=(_NRB,),
    in_specs=[_row_spec, _row_spec, _row_spec, _row_spec,
              _vec_spec, _vec_spec, _vec_spec,
              pl.BlockSpec((1, 1, _RB), lambda i: (i, 0, 0))],
    out_specs=pl.BlockSpec((_G, _D), lambda i: (0, 0)),
    out_shape=jax.ShapeDtypeStruct((_G, _D), jnp.float32),
    scratch_shapes=[pltpu.VMEM((_G, _D), jnp.float32),
                    pltpu.VMEM((_G, _D), jnp.float32)],
)


def kernel(x, edge_index, batch, W1, b1, g1, bt1, W2, b2, g2, bt2,
           W3, b3, g3, bt3):
    src = edge_index[0]
    dst = edge_index[1]
    pad = _EPAD - _E
    lanes = jnp.arange(pad, dtype=jnp.int32)
    src_p = jnp.concatenate(
        [src, lanes % _N]).reshape(_NW, _NCH, _C)
    dst_p = jnp.concatenate(
        [dst, _N + lanes % (_NP - _N)]).reshape(_NW, _NCH, _C)

    dparts = _deg_call(dst_p)

    b1r, g1r, bt1r = b1.reshape(1, _D), g1.reshape(1, _D), bt1.reshape(1, _D)
    b2r, g2r, bt2r = b2.reshape(1, _D), g2.reshape(1, _D), bt2.reshape(1, _D)
    b3r, g3r, bt3r = b3.reshape(1, _D), g3.reshape(1, _D), bt3.reshape(1, _D)
    batch3 = batch.reshape(_NRB, 1, _RB)

    h1, dinv = _lin_call(x, W1, dparts[0, :_N], dparts[1, :_N])
    a = _agg_call(h1, src_p, dst_p)
    h2 = _mid_call(a[0, :_N], a[1, :_N], h1, dinv, b1r, g1r, bt1r, W2)
    a = _agg_call(h2, src_p, dst_p)
    h3 = _mid_call(a[0, :_N], a[1, :_N], h2, dinv, b2r, g2r, bt2r, W3)
    a = _agg_call(h3, src_p, dst_p)
    return _final_call(a[0, :_N], a[1, :_N], h3, dinv, b3r, g3r, bt3r, batch3)
